# Initial kernel scaffold; baseline (speedup 1.0000x reference)
import jax
import jax.numpy as jnp
from jax.experimental import pallas as pl


def kernel(x, embed):
    raise NotImplementedError("write your pallas kernel here")


# speedup vs baseline: 1.0005x; 1.0005x over previous
import functools

import jax
import jax.numpy as jnp
from jax import lax
from jax.experimental import pallas as pl
from jax.experimental.pallas import tpu as pltpu
from jax.experimental.pallas import tpu_sc as plsc

K = 8192
D = 256
N = 65536
TN = 512

NC = 2
NS = 16
NW = NC * NS
BW = N // NW
RCH = 128
NCH = BW // RCH


def _argmax_body(xt_ref, et_ref, e_ref, idx_ref, en_ref):
    @pl.when(pl.program_id(0) == 0)
    def _():
        e = e_ref[...]
        en_ref[...] = jnp.sum(e * e, axis=1).reshape(1, K)

    xt = xt_ref[...]
    m = jnp.dot(xt, et_ref[...], preferred_element_type=jnp.float32)
    xn = jnp.sum(xt * xt, axis=1, keepdims=True)
    dist = -((xn - 2.0 * m) + en_ref[...])

    ks = lax.broadcasted_iota(jnp.int32, (1, K), 1)
    neg = jnp.float32(-jnp.inf)

    def cmax(lo, hi):
        dm = jnp.where((ks >= lo) & (ks < hi), dist, neg)
        return jnp.max(dm, axis=1), jnp.argmax(dm, axis=1)

    v0, i0 = cmax(0, 2736)
    v1, i1 = cmax(2736, 5472)
    v2, i2 = cmax(5472, K)
    accf = v0.astype(jnp.bfloat16).astype(jnp.float32)
    win1 = v1 > accf
    acc_i = jnp.where(win1, i1, i0)
    accf = jnp.where(win1, v1, accf).astype(jnp.bfloat16).astype(jnp.float32)
    win2 = v2 > accf
    idx_ref[...] = jnp.where(win2, i2, acc_i).astype(jnp.int32)


_argmax_call = pl.pallas_call(
    _argmax_body,
    grid=(N // TN,),
    in_specs=[pl.BlockSpec((TN, D), lambda i: (i, 0)),
              pl.BlockSpec((D, K), lambda i: (0, 0)),
              pl.BlockSpec((K, D), lambda i: (0, 0))],
    out_specs=pl.BlockSpec((TN,), lambda i: (i,)),
    out_shape=jax.ShapeDtypeStruct((N,), jnp.int32),
    scratch_shapes=[pltpu.VMEM((1, K), jnp.float32)],
)


def _sc_gather_hist(idx_hbm, embed_hbm, quant_hbm, hist_hbm,
                    idxv, buf, histv, sem):
    cid = lax.axis_index("c")
    sid = lax.axis_index("s")
    wid = sid * NC + cid

    pltpu.sync_copy(idx_hbm.at[pl.ds(wid * NCH, NCH)], idxv)

    def _zero(i, _):
        histv[pl.ds(i * 16, 16)] = jnp.zeros((16,), jnp.int32)
        return 0
    lax.fori_loop(0, K // 16, _zero, 0)

    ones = jnp.ones((16,), jnp.int32)

    def _chunk(c, _):
        pltpu.async_copy(embed_hbm.at[idxv.at[c]], buf, sem).wait()
        pltpu.sync_copy(buf, quant_hbm.at[pl.ds(wid * BW + c * RCH, RCH)])

        def _grp(g, _2):
            iv = idxv[c, pl.ds(g * 16, 16)]
            plsc.addupdate_scatter(histv, [iv], ones)
            return 0
        lax.fori_loop(0, RCH // 16, _grp, 0)
        return 0
    lax.fori_loop(0, NCH, _chunk, 0)

    pltpu.sync_copy(histv, hist_hbm.at[wid])


_sc_call = functools.partial(
    pl.kernel,
    mesh=plsc.VectorSubcoreMesh(core_axis_name="c", subcore_axis_name="s"),
    out_type=[jax.ShapeDtypeStruct((N, D), jnp.float32),
              jax.ShapeDtypeStruct((NW, K), jnp.int32)],
    scratch_types=[pltpu.VMEM((NCH, RCH), jnp.int32),
                   pltpu.VMEM((RCH, D), jnp.float32),
                   pltpu.VMEM((K,), jnp.int32),
                   pltpu.SemaphoreType.DMA],
    compiler_params=pltpu.CompilerParams(needs_layout_passes=False),
)(_sc_gather_hist)


def _perp_body(h_ref, out_ref):
    counts = jnp.sum(h_ref[...].astype(jnp.float32), axis=0, keepdims=True)
    prob = counts * (1.0 / N)
    ent = jnp.sum(prob * jnp.log(prob + 1e-10), axis=1, keepdims=True)
    out_ref[...] = jnp.exp(-ent)


_perp_call = pl.pallas_call(
    _perp_body,
    in_specs=[pl.BlockSpec((NW, K), lambda: (0, 0))],
    out_specs=pl.BlockSpec((1, 1), lambda: (0, 0)),
    out_shape=jax.ShapeDtypeStruct((1, 1), jnp.float32),
)


def kernel(x, embed):
    xf = x.reshape(N, D)
    idx = _argmax_call(xf, embed.T, embed)
    quant, hist = _sc_call(idx.reshape(N // RCH, RCH), embed)
    perp = _perp_call(hist)
    return quant.reshape(x.shape), perp.reshape(())
